# baseline (speedup 1.0000x reference)
import functools

import jax
import jax.numpy as jnp
from jax import lax
from jax.experimental import pallas as pl
from jax.experimental.pallas import tpu as pltpu
from jax.experimental.pallas import tpu_sc as plsc

B = 16384
F = 32
NC = 2
NS = 16
L = 16
NW = NC * NS
BPW = B // NW
CHUNK = 128
NCHUNK = BPW // CHUNK
NGROUP = BPW // L
REG_C = 0.1
INV_LN2 = 1.4426950408889634


def _make_sc_bpr(interpret=False):
    return functools.partial(
        pl.kernel,
        mesh=plsc.VectorSubcoreMesh(core_axis_name="c", subcore_axis_name="s"),
        compiler_params=pltpu.CompilerParams(
            needs_layout_passes=False, use_tc_tiling_on_sc=False),
        interpret=interpret,
        out_type=[
            jax.ShapeDtypeStruct((B,), jnp.float32),
            jax.ShapeDtypeStruct((B,), jnp.float32),
            jax.ShapeDtypeStruct((NW * L,), jnp.float32),
        ],
        scratch_types=[
            pltpu.VMEM((BPW,), jnp.int32),
            pltpu.VMEM((BPW,), jnp.int32),
            pltpu.VMEM((BPW,), jnp.int32),
            pltpu.VMEM((F, BPW), jnp.float32),
            pltpu.VMEM((F, BPW), jnp.float32),
            pltpu.VMEM((F, BPW), jnp.float32),
            pltpu.VMEM((BPW,), jnp.float32),
            pltpu.VMEM((BPW,), jnp.float32),
            pltpu.VMEM((L,), jnp.float32),
            pltpu.SemaphoreType.DMA,
        ],
    )(_sc_bpr_body)


def _sc_bpr_body(Ut_hbm, Vt_hbm, u_hbm, i_hbm, j_hbm,
            yui_hbm, yuj_hbm, regp_hbm,
            idx_u, idx_i, idx_j, cols_u, cols_i, cols_j,
            yui_v, yuj_v, regp_v, sem):
    wid = lax.axis_index("s") * NC + lax.axis_index("c")
    base = wid * BPW

    pltpu.sync_copy(u_hbm.at[pl.ds(base, BPW)], idx_u)
    pltpu.sync_copy(i_hbm.at[pl.ds(base, BPW)], idx_i)
    pltpu.sync_copy(j_hbm.at[pl.ds(base, BPW)], idx_j)

    copies = []
    for f in range(F):
        for c in range(NCHUNK):
            isl = pl.ds(c * CHUNK, CHUNK)
            copies.append(pltpu.async_copy(
                Ut_hbm.at[f].at[idx_u.at[isl]], cols_u.at[f].at[isl], sem))
            copies.append(pltpu.async_copy(
                Vt_hbm.at[f].at[idx_i.at[isl]], cols_i.at[f].at[isl], sem))
            copies.append(pltpu.async_copy(
                Vt_hbm.at[f].at[idx_j.at[isl]], cols_j.at[f].at[isl], sem))
    for cp in copies:
        cp.wait()

    def group_body(g, reg_acc):
        gsl = pl.ds(g * L, L)

        def factor_body(f, carry):
            acc_ui, acc_uj, reg = carry
            uu = cols_u[f, gsl]
            vi = cols_i[f, gsl]
            vj = cols_j[f, gsl]
            return (acc_ui + uu * vi, acc_uj + uu * vj,
                    reg + (uu * uu + (vi * vi + vj * vj)))

        z = jnp.zeros((L,), jnp.float32)
        acc_ui, acc_uj, reg_acc = lax.fori_loop(0, F, factor_body,
                                                (z, z, reg_acc))
        yui_v[gsl] = acc_ui
        yuj_v[gsl] = acc_uj
        return reg_acc

    reg_acc = lax.fori_loop(0, NGROUP, group_body, jnp.zeros((L,), jnp.float32))
    regp_v[...] = reg_acc

    pltpu.sync_copy(yui_v, yui_hbm.at[pl.ds(base, BPW)])
    pltpu.sync_copy(yuj_v, yuj_hbm.at[pl.ds(base, BPW)])
    pltpu.sync_copy(regp_v, regp_hbm.at[pl.ds(wid * L, L)])


_sc_bpr = _make_sc_bpr()


def _loss_body(yui_ref, yuj_ref, regp_ref, out_ref):
    d = yui_ref[...] - yuj_ref[...]
    nls = jnp.log(1.0 + jnp.exp(-d)) * INV_LN2
    out_ref[0, 0] = REG_C * jnp.sum(regp_ref[...]) + jnp.sum(nls)


_loss_call = pl.pallas_call(
    _loss_body,
    out_shape=jax.ShapeDtypeStruct((1, 1), jnp.float32),
    out_specs=pl.BlockSpec(memory_space=pltpu.SMEM),
)


def kernel(U, V, u, i, j):
    y_ui, y_uj, regp = _sc_bpr(U.T, V.T, u, i, j)
    loss = _loss_call(y_ui.reshape(B // 128, 128), y_uj.reshape(B // 128, 128),
                      regp.reshape(NW * L // 128, 128))
    return y_ui, y_uj, loss.reshape(())


# speedup vs baseline: 2.8670x; 2.8670x over previous
import functools

import jax
import jax.numpy as jnp
from jax import lax
from jax.experimental import pallas as pl
from jax.experimental.pallas import tpu as pltpu
from jax.experimental.pallas import tpu_sc as plsc

B = 16384
F = 32
NC = 2
NS = 16
L = 16
NW = NC * NS
BPW = B // NW
CHUNK = 128
NCHUNK = BPW // CHUNK
REG_C = 0.1
INV_LN2 = 1.4426950408889634


@functools.partial(
    pl.kernel,
    mesh=plsc.VectorSubcoreMesh(core_axis_name="c", subcore_axis_name="s"),
    compiler_params=pltpu.CompilerParams(
        needs_layout_passes=False, use_tc_tiling_on_sc=False),
    out_type=[
        jax.ShapeDtypeStruct((B,), jnp.float32),
        jax.ShapeDtypeStruct((B,), jnp.float32),
        jax.ShapeDtypeStruct((NW, L), jnp.float32),
    ],
    scratch_types=[
        pltpu.VMEM((NCHUNK, CHUNK), jnp.int32),
        pltpu.VMEM((NCHUNK, CHUNK), jnp.int32),
        pltpu.VMEM((NCHUNK, CHUNK), jnp.int32),
        pltpu.VMEM((BPW, F), jnp.float32),
        pltpu.VMEM((BPW, F), jnp.float32),
        pltpu.VMEM((BPW, F), jnp.float32),
        pltpu.VMEM((BPW,), jnp.float32),
        pltpu.VMEM((BPW,), jnp.float32),
        pltpu.VMEM((L,), jnp.float32),
        pltpu.SemaphoreType.DMA,
    ],
)
def _sc_bpr(U_hbm, V_hbm, u_hbm, i_hbm, j_hbm,
            yui_hbm, yuj_hbm, regp_hbm,
            idx_u, idx_i, idx_j, rows_u, rows_i, rows_j,
            yui_v, yuj_v, regp_v, sem):
    wid = lax.axis_index("s") * NC + lax.axis_index("c")
    base = wid * BPW
    crow = wid * NCHUNK

    pltpu.sync_copy(u_hbm.at[pl.ds(crow, NCHUNK)], idx_u)
    pltpu.sync_copy(i_hbm.at[pl.ds(crow, NCHUNK)], idx_i)
    pltpu.sync_copy(j_hbm.at[pl.ds(crow, NCHUNK)], idx_j)

    copies = []
    for c in range(NCHUNK):
        sl = pl.ds(c * CHUNK, CHUNK)
        copies.append(pltpu.async_copy(U_hbm.at[idx_u.at[c]], rows_u.at[sl], sem))
        copies.append(pltpu.async_copy(V_hbm.at[idx_i.at[c]], rows_i.at[sl], sem))
        copies.append(pltpu.async_copy(V_hbm.at[idx_j.at[c]], rows_j.at[sl], sem))
    for cp in copies:
        cp.wait()

    lane = lax.iota(jnp.int32, L)
    last = lane == (L - 1)

    def row_body(r, reg_acc):
        u0 = rows_u[r, pl.ds(0, L)]
        u1 = rows_u[r, pl.ds(L, L)]
        vi0 = rows_i[r, pl.ds(0, L)]
        vi1 = rows_i[r, pl.ds(L, L)]
        vj0 = rows_j[r, pl.ds(0, L)]
        vj1 = rows_j[r, pl.ds(L, L)]
        cum_ui = plsc.cumsum(u0 * vi0 + u1 * vi1)
        cum_uj = plsc.cumsum(u0 * vj0 + u1 * vj1)
        ridx = jnp.full((L,), 0, jnp.int32) + r
        plsc.store_scatter(yui_v, [ridx], cum_ui, mask=last)
        plsc.store_scatter(yuj_v, [ridx], cum_uj, mask=last)
        return reg_acc + ((u0 * u0 + u1 * u1)
                          + (vi0 * vi0 + vi1 * vi1)
                          + (vj0 * vj0 + vj1 * vj1))

    reg_acc = lax.fori_loop(0, BPW, row_body, jnp.zeros((L,), jnp.float32))
    regp_v[...] = reg_acc

    pltpu.sync_copy(yui_v, yui_hbm.at[pl.ds(base, BPW)])
    pltpu.sync_copy(yuj_v, yuj_hbm.at[pl.ds(base, BPW)])
    pltpu.sync_copy(regp_v, regp_hbm.at[wid])


def _loss_body(yui_ref, yuj_ref, regp_ref, out_ref):
    d = yui_ref[...] - yuj_ref[...]
    nls = jnp.log(1.0 + jnp.exp(-d)) * INV_LN2
    out_ref[0, 0] = REG_C * jnp.sum(regp_ref[...]) + jnp.sum(nls)


_loss_call = pl.pallas_call(
    _loss_body,
    out_shape=jax.ShapeDtypeStruct((1, 1), jnp.float32),
    out_specs=pl.BlockSpec(memory_space=pltpu.SMEM),
)


NROW = 1000000
BLK = 128
NBLK_FULL = NROW // BLK
NBLK = NBLK_FULL + 1
TAIL_C0 = NBLK_FULL * BLK
PER_W = (NBLK + NW - 1) // NW


@functools.partial(
    pl.kernel,
    mesh=plsc.VectorSubcoreMesh(core_axis_name="c", subcore_axis_name="s"),
    compiler_params=pltpu.CompilerParams(
        needs_layout_passes=False, use_tc_tiling_on_sc=True),
    out_type=[
        jax.ShapeDtypeStruct((NROW * F,), jnp.float32),
        jax.ShapeDtypeStruct((NROW * F,), jnp.float32),
    ],
    scratch_types=[
        pltpu.VMEM((F, BLK), jnp.float32),
        pltpu.VMEM((F * BLK,), jnp.float32),
        pltpu.SemaphoreType.DMA,
    ],
)
def _sc_detile(Ut_hbm, Vt_hbm, uf_hbm, vf_hbm, blk, obuf, sem):
    wid = lax.axis_index("s") * NC + lax.axis_index("c")
    lane32 = lax.iota(jnp.int32, L) * F

    def do_table(t_hbm, o_hbm):
        def blk_body(k, _):
            b = wid + k * NW

            @pl.when(b < NBLK)
            def _():
                is_tail = b >= NBLK_FULL
                c0 = jnp.where(is_tail, TAIL_C0, b * BLK)
                pltpu.sync_copy(t_hbm.at[:, pl.ds(c0, BLK)], blk)
                for f in range(F):
                    for rg in range(BLK // L):
                        vals = blk[f, pl.ds(rg * L, L)]
                        tgt = lane32 + (rg * L * F + f)
                        plsc.store_scatter(obuf, [tgt], vals)

                @pl.when(jnp.logical_not(is_tail))
                def _():
                    pltpu.sync_copy(obuf, o_hbm.at[pl.ds(c0 * F, BLK * F)])

                @pl.when(is_tail)
                def _():
                    pltpu.sync_copy(obuf.at[pl.ds(0, (NROW - TAIL_C0) * F)],
                                    o_hbm.at[pl.ds(TAIL_C0 * F,
                                                   (NROW - TAIL_C0) * F)])
            return 0

        lax.fori_loop(0, PER_W, blk_body, 0)

    do_table(Ut_hbm, uf_hbm)
    do_table(Vt_hbm, vf_hbm)


def kernel(U, V, u, i, j):
    uf, vf = _sc_detile(U.T, V.T)
    u2 = u.reshape(B // CHUNK, CHUNK)
    i2 = i.reshape(B // CHUNK, CHUNK)
    j2 = j.reshape(B // CHUNK, CHUNK)
    y_ui, y_uj, regp = _sc_bpr(uf.reshape(NROW, F), vf.reshape(NROW, F),
                               u2, i2, j2)
    loss = _loss_call(y_ui.reshape(B // 128, 128), y_uj.reshape(B // 128, 128),
                      regp)
    return y_ui, y_uj, loss.reshape(())


# speedup vs baseline: 3.3325x; 1.1624x over previous
import functools

import jax
import jax.numpy as jnp
from jax import lax
from jax.experimental import pallas as pl
from jax.experimental.pallas import tpu as pltpu
from jax.experimental.pallas import tpu_sc as plsc

B = 16384
F = 32
NC = 2
NS = 16
L = 16
NW = NC * NS
BPW = B // NW
CHUNK = 128
NCHUNK = BPW // CHUNK
REG_C = 0.1
INV_LN2 = 1.4426950408889634


@functools.partial(
    pl.kernel,
    mesh=plsc.VectorSubcoreMesh(core_axis_name="c", subcore_axis_name="s"),
    compiler_params=pltpu.CompilerParams(
        needs_layout_passes=False, use_tc_tiling_on_sc=False),
    out_type=[
        jax.ShapeDtypeStruct((B,), jnp.float32),
        jax.ShapeDtypeStruct((B,), jnp.float32),
        jax.ShapeDtypeStruct((NW, L), jnp.float32),
    ],
    scratch_types=[
        pltpu.VMEM((NCHUNK, CHUNK), jnp.int32),
        pltpu.VMEM((NCHUNK, CHUNK), jnp.int32),
        pltpu.VMEM((NCHUNK, CHUNK), jnp.int32),
        pltpu.VMEM((BPW, F), jnp.float32),
        pltpu.VMEM((BPW, F), jnp.float32),
        pltpu.VMEM((BPW, F), jnp.float32),
        pltpu.VMEM((BPW,), jnp.float32),
        pltpu.VMEM((BPW,), jnp.float32),
        pltpu.VMEM((L,), jnp.float32),
        pltpu.SemaphoreType.DMA,
    ],
)
def _sc_bpr(U_hbm, V_hbm, u_hbm, i_hbm, j_hbm,
            yui_hbm, yuj_hbm, regp_hbm,
            idx_u, idx_i, idx_j, rows_u, rows_i, rows_j,
            yui_v, yuj_v, regp_v, sem):
    wid = lax.axis_index("s") * NC + lax.axis_index("c")
    base = wid * BPW
    crow = wid * NCHUNK

    pltpu.sync_copy(u_hbm.at[pl.ds(crow, NCHUNK)], idx_u)
    pltpu.sync_copy(i_hbm.at[pl.ds(crow, NCHUNK)], idx_i)
    pltpu.sync_copy(j_hbm.at[pl.ds(crow, NCHUNK)], idx_j)

    copies = []
    for c in range(NCHUNK):
        sl = pl.ds(c * CHUNK, CHUNK)
        copies.append(pltpu.async_copy(U_hbm.at[idx_u.at[c]], rows_u.at[sl], sem))
        copies.append(pltpu.async_copy(V_hbm.at[idx_i.at[c]], rows_i.at[sl], sem))
        copies.append(pltpu.async_copy(V_hbm.at[idx_j.at[c]], rows_j.at[sl], sem))
    for cp in copies:
        cp.wait()

    lane = lax.iota(jnp.int32, L)
    last = lane == (L - 1)

    def row_body(r, reg_acc):
        u0 = rows_u[r, pl.ds(0, L)]
        u1 = rows_u[r, pl.ds(L, L)]
        vi0 = rows_i[r, pl.ds(0, L)]
        vi1 = rows_i[r, pl.ds(L, L)]
        vj0 = rows_j[r, pl.ds(0, L)]
        vj1 = rows_j[r, pl.ds(L, L)]
        cum_ui = plsc.cumsum(u0 * vi0 + u1 * vi1)
        cum_uj = plsc.cumsum(u0 * vj0 + u1 * vj1)
        ridx = jnp.full((L,), 0, jnp.int32) + r
        plsc.store_scatter(yui_v, [ridx], cum_ui, mask=last)
        plsc.store_scatter(yuj_v, [ridx], cum_uj, mask=last)
        return reg_acc + ((u0 * u0 + u1 * u1)
                          + (vi0 * vi0 + vi1 * vi1)
                          + (vj0 * vj0 + vj1 * vj1))

    reg_acc = lax.fori_loop(0, BPW, row_body, jnp.zeros((L,), jnp.float32))
    regp_v[...] = reg_acc

    pltpu.sync_copy(yui_v, yui_hbm.at[pl.ds(base, BPW)])
    pltpu.sync_copy(yuj_v, yuj_hbm.at[pl.ds(base, BPW)])
    pltpu.sync_copy(regp_v, regp_hbm.at[wid])


def _loss_body(yui_ref, yuj_ref, regp_ref, out_ref):
    d = yui_ref[...] - yuj_ref[...]
    nls = jnp.log(1.0 + jnp.exp(-d)) * INV_LN2
    out_ref[0, 0] = REG_C * jnp.sum(regp_ref[...]) + jnp.sum(nls)


_loss_call = pl.pallas_call(
    _loss_body,
    out_shape=jax.ShapeDtypeStruct((1, 1), jnp.float32),
    out_specs=pl.BlockSpec(memory_space=pltpu.SMEM),
)


NROW = 1000000
BLK = 512
NBLK_FULL = NROW // BLK
NBLK = NBLK_FULL + 1
TAIL_C0 = NBLK_FULL * BLK
TAIL_R = 64
PER_W = (NBLK + NW - 1) // NW


@functools.partial(
    pl.kernel,
    mesh=plsc.VectorSubcoreMesh(core_axis_name="c", subcore_axis_name="s"),
    compiler_params=pltpu.CompilerParams(
        needs_layout_passes=False, use_tc_tiling_on_sc=True),
    out_type=[
        jax.ShapeDtypeStruct((NROW * F,), jnp.float32),
        jax.ShapeDtypeStruct((NROW * F,), jnp.float32),
    ],
    scratch_types=[
        pltpu.VMEM((F, BLK), jnp.float32),
        pltpu.VMEM((F, 128), jnp.float32),
        pltpu.VMEM((F * BLK,), jnp.float32),
        pltpu.SemaphoreType.DMA,
    ],
)
def _sc_detile(Ut_hbm, Vt_hbm, uf_hbm, vf_hbm, blk, blkt, obuf, sem):
    wid = lax.axis_index("s") * NC + lax.axis_index("c")
    lane32 = lax.iota(jnp.int32, L) * F

    def do_table(t_hbm, o_hbm):
        def blk_body(k, _):
            b = wid + k * NW

            @pl.when(b < NBLK_FULL)
            def _():
                c0 = b * BLK
                pltpu.sync_copy(t_hbm.at[:, pl.ds(c0, BLK)], blk)
                for f in range(F):
                    for rg in range(BLK // L):
                        vals = blk[f, pl.ds(rg * L, L)]
                        tgt = lane32 + (rg * L * F + f)
                        plsc.store_scatter(obuf, [tgt], vals)
                pltpu.sync_copy(obuf, o_hbm.at[pl.ds(c0 * F, BLK * F)])

            @pl.when(b == NBLK_FULL)
            def _():
                c0 = jnp.where(b == NBLK_FULL, TAIL_C0, 0)
                pltpu.sync_copy(t_hbm.at[:, pl.ds(c0, 128)], blkt)
                for f in range(F):
                    for rg in range((NROW - TAIL_C0) // L):
                        vals = blkt[f, pl.ds(rg * L, L)]
                        tgt = lane32 + (rg * L * F + f)
                        plsc.store_scatter(obuf, [tgt], vals)
                pltpu.sync_copy(obuf.at[pl.ds(0, (NROW - TAIL_C0) * F)],
                                o_hbm.at[pl.ds(TAIL_C0 * F,
                                               (NROW - TAIL_C0) * F)])
            return 0

        lax.fori_loop(0, PER_W, blk_body, 0)

    do_table(Ut_hbm, uf_hbm)
    do_table(Vt_hbm, vf_hbm)


def kernel(U, V, u, i, j):
    uf, vf = _sc_detile(U.T, V.T)
    u2 = u.reshape(B // CHUNK, CHUNK)
    i2 = i.reshape(B // CHUNK, CHUNK)
    j2 = j.reshape(B // CHUNK, CHUNK)
    y_ui, y_uj, regp = _sc_bpr(uf.reshape(NROW, F), vf.reshape(NROW, F),
                               u2, i2, j2)
    loss = _loss_call(y_ui.reshape(B // 128, 128), y_uj.reshape(B // 128, 128),
                      regp)
    return y_ui, y_uj, loss.reshape(())
